# baseline (speedup 1.0000x reference)
import functools

import jax
import jax.numpy as jnp
from jax import lax
from jax.experimental import pallas as pl
from jax.experimental.pallas import tpu as pltpu
from jax.experimental.pallas import tpu_sc as plsc

V = 1000000
H = V // 2
B = 16384
D = 64
K = 3
NC = 2
NS = 16
NW = NC * NS
BPW = B // NW
CH = 64
NCH = BPW // CH
L = 16
PIECES = D // L
GPR = 8

CROWS = 2000
CGRID = H // CROWS


def _compact_body(x_lo, x_hi, x_out):
    x_out[:, :D] = x_lo[...]
    x_out[:, D:] = x_hi[...]


_compact = pl.pallas_call(
    _compact_body,
    grid=(CGRID,),
    in_specs=[
        pl.BlockSpec((CROWS, D), lambda i: (i, 0)),
        pl.BlockSpec((CROWS, D), lambda i: (i + CGRID, 0)),
    ],
    out_specs=pl.BlockSpec((CROWS, 2 * D), lambda i: (i, 0)),
    out_shape=jax.ShapeDtypeStruct((H, 2 * D), jnp.float32),
)

_mesh = plsc.VectorSubcoreMesh(core_axis_name="c", subcore_axis_name="s")


@functools.partial(
    pl.kernel,
    mesh=_mesh,
    out_type=[
        jax.ShapeDtypeStruct((B // GPR, 128), jnp.float32),
        jax.ShapeDtypeStruct((K * B // GPR, 128), jnp.float32),
    ],
    scratch_types=[
        pltpu.VMEM((NCH, CH), jnp.int32),
        pltpu.VMEM((NCH, CH), jnp.int32),
        pltpu.VMEM((K * NCH, CH), jnp.int32),
        pltpu.VMEM((NCH, CH), jnp.int32),
        pltpu.VMEM((NCH, CH), jnp.int32),
        pltpu.VMEM((K * NCH, CH), jnp.int32),
        pltpu.VMEM((CH, 128), jnp.float32),
        pltpu.VMEM((CH, 128), jnp.float32),
        pltpu.VMEM((K, CH, 128), jnp.float32),
        pltpu.VMEM((BPW // GPR, 128), jnp.float32),
        pltpu.VMEM((K, BPW // GPR, 128), jnp.float32),
        pltpu.SemaphoreType.DMA,
    ],
)
def _sc_scores(center_hbm, context_hbm, negt_hbm, ctab_hbm, xtab_hbm,
               pos_out, neg_out,
               cidx, xidx, nidx, cpr, xpr, npr,
               crows, xrows, nrows, pbuf, nbuf, sem):
    wid = lax.axis_index("s") * NC + lax.axis_index("c")
    base = wid * BPW

    for j in range(NCH):
        pltpu.sync_copy(center_hbm.at[pl.ds(base + j * CH, CH)], cidx.at[j])
        pltpu.sync_copy(context_hbm.at[pl.ds(base + j * CH, CH)], xidx.at[j])
        for k in range(K):
            pltpu.sync_copy(negt_hbm.at[pl.ds(k * B + base + j * CH, CH)],
                            nidx.at[k * NCH + j])

    for j in range(NCH):
        for t in range(CH // L):
            s = pl.ds(t * L, L)
            cpr[j, s] = cidx[j, s] >> 1
            xpr[j, s] = jnp.where(xidx[j, s] >= H, xidx[j, s] - H,
                                  xidx[j, s])
            for k in range(K):
                nv = nidx[k * NCH + j, s]
                npr[k * NCH + j, s] = jnp.where(nv >= H, nv - H, nv)

    for j in range(NCH):
        cps = [
            pltpu.async_copy(ctab_hbm.at[cpr.at[j]], crows, sem),
            pltpu.async_copy(xtab_hbm.at[xpr.at[j]], xrows, sem),
        ]
        for k in range(K):
            cps.append(pltpu.async_copy(xtab_hbm.at[npr.at[k * NCH + j]],
                                        nrows.at[k], sem))
        for cp in cps:
            cp.wait()

        def body(t, carry, j=j):
            blk = pl.ds(t * L, L)
            coffv = (cidx[j, blk] & 1) * D
            xoffv = jnp.where(xidx[j, blk] >= H, D, 0)
            noffv = [jnp.where(nidx[k * NCH + j, blk] >= H, D, 0)
                     for k in range(K)]
            for r in range(L):
                e = t * L + r
                row = j * (CH // GPR) + t * (L // GPR) + r // GPR
                coff = coffv[r]
                xoff = xoffv[r]
                cs = [crows[e, pl.ds(coff + p * L, L)] for p in range(PIECES)]
                xs = [xrows[e, pl.ds(xoff + p * L, L)] for p in range(PIECES)]
                pv = (cs[0] * xs[0] + cs[1] * xs[1]
                      + cs[2] * xs[2] + cs[3] * xs[3])
                pbuf[row, pl.ds((r % GPR) * L, L)] = pv
                for k in range(K):
                    noff = noffv[k][r]
                    ns = [nrows[k, e, pl.ds(noff + p * L, L)]
                          for p in range(PIECES)]
                    nv = (cs[0] * ns[0] + cs[1] * ns[1]
                          + cs[2] * ns[2] + cs[3] * ns[3])
                    nbuf[k, row, pl.ds((r % GPR) * L, L)] = nv
            return carry

        lax.fori_loop(0, CH // L, body, 0)

    pltpu.sync_copy(
        pbuf,
        pos_out.at[pl.ds(pl.multiple_of(base // GPR, 8), BPW // GPR)])
    for k in range(K):
        pltpu.sync_copy(
            nbuf.at[k],
            neg_out.at[pl.ds(pl.multiple_of((k * B + base) // GPR, 8),
                             BPW // GPR)])


def _loss_body(pos_ref, neg_ref, out_ref):
    pos = jnp.sum(pos_ref[...].reshape(B // GPR, GPR, L), axis=2)
    neg = jnp.sum(neg_ref[...].reshape(K * B // GPR, GPR, L), axis=2)

    def softplus(z):
        return jnp.maximum(z, 0.0) + jnp.log1p(jnp.exp(-jnp.abs(z)))

    total = (jnp.sum(softplus(-pos)) + jnp.sum(softplus(neg))) / B
    out_ref[...] = jnp.reshape(total, (1, 1))


_loss = pl.pallas_call(
    _loss_body,
    out_shape=jax.ShapeDtypeStruct((1, 1), jnp.float32),
)


def kernel(center, context, negative_samples, center_table, context_table):
    center = center.astype(jnp.int32)
    context = context.astype(jnp.int32)
    negt = negative_samples.astype(jnp.int32).T.reshape(-1)
    ctab2 = center_table.reshape(H, 2 * D)
    xtab2 = _compact(context_table, context_table)
    pos_pv, neg_pv = _sc_scores(center, context, negt, ctab2, xtab2)
    loss = _loss(pos_pv, neg_pv)
    return loss[0, 0]


# speedup vs baseline: 1.0668x; 1.0668x over previous
import functools

import jax
import jax.numpy as jnp
from jax import lax
from jax.experimental import pallas as pl
from jax.experimental.pallas import tpu as pltpu
from jax.experimental.pallas import tpu_sc as plsc

B = 16384
D = 64
K = 3
NC = 2
NS = 16
NW = NC * NS
BPW = B // NW
CH = 64
NCH = BPW // CH
L = 16
PIECES = D // L
GPR = 8

_mesh = plsc.VectorSubcoreMesh(core_axis_name="c", subcore_axis_name="s")


@functools.partial(
    pl.kernel,
    mesh=_mesh,
    out_type=[
        jax.ShapeDtypeStruct((B // GPR, 128), jnp.float32),
        jax.ShapeDtypeStruct((K * B // GPR, 128), jnp.float32),
    ],
    scratch_types=[
        pltpu.VMEM((NCH, CH), jnp.int32),
        pltpu.VMEM((NCH, CH), jnp.int32),
        pltpu.VMEM((K * NCH, CH), jnp.int32),
        pltpu.VMEM((NCH, CH), jnp.int32),
        pltpu.VMEM((NCH, CH), jnp.int32),
        pltpu.VMEM((K * NCH, CH), jnp.int32),
        pltpu.VMEM((CH, 128), jnp.float32),
        pltpu.VMEM((CH, 128), jnp.float32),
        pltpu.VMEM((K, CH, 128), jnp.float32),
        pltpu.VMEM((BPW // GPR, 128), jnp.float32),
        pltpu.VMEM((K, BPW // GPR, 128), jnp.float32),
        pltpu.SemaphoreType.DMA,
    ],
)
def _sc_scores(center_hbm, context_hbm, negt_hbm, ctab_hbm, xtab_hbm,
               pos_out, neg_out,
               cidx, xidx, nidx, cpr, xpr, npr,
               crows, xrows, nrows, pbuf, nbuf, sem):
    wid = lax.axis_index("s") * NC + lax.axis_index("c")
    base = wid * BPW

    for j in range(NCH):
        pltpu.sync_copy(center_hbm.at[pl.ds(base + j * CH, CH)], cidx.at[j])
        pltpu.sync_copy(context_hbm.at[pl.ds(base + j * CH, CH)], xidx.at[j])
        for k in range(K):
            pltpu.sync_copy(negt_hbm.at[pl.ds(k * B + base + j * CH, CH)],
                            nidx.at[k * NCH + j])

    for j in range(NCH):
        for t in range(CH // L):
            s = pl.ds(t * L, L)
            cpr[j, s] = cidx[j, s] >> 1
            xpr[j, s] = xidx[j, s] >> 1
            for k in range(K):
                npr[k * NCH + j, s] = nidx[k * NCH + j, s] >> 1

    for j in range(NCH):
        cps = [
            pltpu.async_copy(ctab_hbm.at[cpr.at[j]], crows, sem),
            pltpu.async_copy(xtab_hbm.at[xpr.at[j]], xrows, sem),
        ]
        for k in range(K):
            cps.append(pltpu.async_copy(xtab_hbm.at[npr.at[k * NCH + j]],
                                        nrows.at[k], sem))
        for cp in cps:
            cp.wait()

        def body(t, carry, j=j):
            blk = pl.ds(t * L, L)
            coffv = (cidx[j, blk] & 1) * D
            xoffv = (xidx[j, blk] & 1) * D
            noffv = [(nidx[k * NCH + j, blk] & 1) * D for k in range(K)]
            for r in range(L):
                e = t * L + r
                row = j * (CH // GPR) + t * (L // GPR) + r // GPR
                coff = coffv[r]
                xoff = xoffv[r]
                cs = [crows[e, pl.ds(coff + p * L, L)] for p in range(PIECES)]
                xs = [xrows[e, pl.ds(xoff + p * L, L)] for p in range(PIECES)]
                pv = (cs[0] * xs[0] + cs[1] * xs[1]
                      + cs[2] * xs[2] + cs[3] * xs[3])
                pbuf[row, pl.ds((r % GPR) * L, L)] = pv
                for k in range(K):
                    noff = noffv[k][r]
                    ns = [nrows[k, e, pl.ds(noff + p * L, L)]
                          for p in range(PIECES)]
                    nv = (cs[0] * ns[0] + cs[1] * ns[1]
                          + cs[2] * ns[2] + cs[3] * ns[3])
                    nbuf[k, row, pl.ds((r % GPR) * L, L)] = nv
            return carry

        lax.fori_loop(0, CH // L, body, 0)

    pltpu.sync_copy(
        pbuf,
        pos_out.at[pl.ds(pl.multiple_of(base // GPR, 8), BPW // GPR)])
    for k in range(K):
        pltpu.sync_copy(
            nbuf.at[k],
            neg_out.at[pl.ds(pl.multiple_of((k * B + base) // GPR, 8),
                             BPW // GPR)])


def _loss_body(pos_ref, neg_ref, out_ref):
    pos = jnp.sum(pos_ref[...].reshape(B // GPR, GPR, L), axis=2)
    neg = jnp.sum(neg_ref[...].reshape(K * B // GPR, GPR, L), axis=2)

    def softplus(z):
        return jnp.maximum(z, 0.0) + jnp.log1p(jnp.exp(-jnp.abs(z)))

    total = (jnp.sum(softplus(-pos)) + jnp.sum(softplus(neg))) / B
    out_ref[...] = jnp.reshape(total, (1, 1))


_loss = pl.pallas_call(
    _loss_body,
    out_shape=jax.ShapeDtypeStruct((1, 1), jnp.float32),
)


def kernel(center, context, negative_samples, center_table, context_table):
    center = center.astype(jnp.int32)
    context = context.astype(jnp.int32)
    negt = negative_samples.astype(jnp.int32).T.reshape(-1)
    ctab2 = center_table.reshape(center_table.shape[0] // 2, 2 * D)
    xtab2 = context_table.reshape(context_table.shape[0] // 2, 2 * D)
    pos_pv, neg_pv = _sc_scores(center, context, negt, ctab2, xtab2)
    loss = _loss(pos_pv, neg_pv)
    return loss[0, 0]
